# Initial kernel scaffold; baseline (speedup 1.0000x reference)
import functools

import jax
import jax.numpy as jnp
from jax import lax
from jax.experimental import pallas as pl
from jax.experimental.pallas import tpu as pltpu
from jax.experimental.pallas import tpu_sc as plsc

B, C, H, W = 16, 34, 128, 128
K = 32
HW = H * W
L = 16
KV = K // L
NPT = K * C
CHUNK = 128
NCH = NPT // CHUNK
REM = NPT - NCH * CHUNK


def _body(out_hbm, ind_hbm, m_hbm, t_hbm, res_hbm,
          ind_v, idx_v, pred_v, m_v, t_v, part_v, all_v, out_v, shared, sem):
    b = lax.axis_index("s")
    row0 = b * K

    pltpu.sync_copy(ind_hbm.at[pl.ds(row0, K)], ind_v)
    pltpu.sync_copy(m_hbm.at[pl.ds(row0 * C, NPT)], m_v)
    pltpu.sync_copy(t_hbm.at[pl.ds(row0 * C, NPT)], t_v)

    base_b = b * (C * HW)
    inds = [ind_v[pl.ds(kv * L, L)] for kv in range(KV)]
    for c in range(C):
        off = base_b + c * HW
        for kv in range(KV):
            idx_v[pl.ds((kv * C + c) * L, L)] = inds[kv] + off

    copies = []
    for j in range(NCH):
        copies.append(pltpu.async_copy(
            out_hbm.at[idx_v.at[pl.ds(j * CHUNK, CHUNK)]],
            pred_v.at[pl.ds(j * CHUNK, CHUNK)], sem))
    copies.append(pltpu.async_copy(
        out_hbm.at[idx_v.at[pl.ds(NCH * CHUNK, REM)]],
        pred_v.at[pl.ds(NCH * CHUNK, REM)], sem))
    for cp in copies:
        cp.wait()

    lane_c = jnp.arange(L, dtype=jnp.int32) * C
    acc = jnp.zeros((L,), jnp.float32)
    mac = jnp.zeros((L,), jnp.float32)
    for kv in range(KV):
        kbase = kv * L * C
        for c in range(C):
            p = pred_v[pl.ds((kv * C + c) * L, L)]
            kidx = lane_c + (kbase + c)
            m = plsc.load_gather(m_v, [kidx])
            tg = plsc.load_gather(t_v, [kidx])
            acc = acc + jnp.abs(p * m - tg * m)
            mac = mac + m
    part_v[pl.ds(0, L)] = acc
    part_v[pl.ds(L, L)] = mac

    pltpu.sync_copy(part_v, shared.at[b])
    plsc.subcore_barrier()

    @pl.when(b == 0)
    def _():
        pltpu.sync_copy(shared, all_v)
        a = jnp.zeros((L,), jnp.float32)
        m = jnp.zeros((L,), jnp.float32)
        for i in range(B):
            a = a + all_v[pl.ds(i * 2 * L, L)]
            m = m + all_v[pl.ds(i * 2 * L + L, L)]
        loss = jnp.sum(a) / (jnp.sum(m) + 0.0001)
        out_v[...] = jnp.full((L,), loss, jnp.float32)
        pltpu.sync_copy(out_v, res_hbm)


@jax.jit
def kernel(output, mask, ind, target):
    out_flat = output.reshape(-1)
    ind_flat = ind.astype(jnp.int32).reshape(-1)
    m_flat = mask.astype(jnp.float32).reshape(-1)
    t_flat = target.astype(jnp.float32).reshape(-1)

    mesh = plsc.VectorSubcoreMesh(
        core_axis_name="c", subcore_axis_name="s", num_cores=1)
    run = functools.partial(
        pl.kernel,
        out_type=jax.ShapeDtypeStruct((L,), jnp.float32),
        mesh=mesh,
        scratch_types=[
            pltpu.VMEM((K,), jnp.int32),
            pltpu.VMEM((NPT,), jnp.int32),
            pltpu.VMEM((NPT,), jnp.float32),
            pltpu.VMEM((NPT,), jnp.float32),
            pltpu.VMEM((NPT,), jnp.float32),
            pltpu.VMEM((2 * L,), jnp.float32),
            pltpu.VMEM((B * 2 * L,), jnp.float32),
            pltpu.VMEM((L,), jnp.float32),
            pltpu.VMEM_SHARED((B, 2 * L), jnp.float32),
            pltpu.SemaphoreType.DMA,
        ],
    )(_body)
    res = run(out_flat, ind_flat, m_flat, t_flat)
    return res[0]


# speedup vs baseline: 3.0947x; 3.0947x over previous
import functools

import jax
import jax.numpy as jnp
from jax import lax
from jax.experimental import pallas as pl
from jax.experimental.pallas import tpu as pltpu
from jax.experimental.pallas import tpu_sc as plsc

B, C, H, W = 16, 34, 128, 128
K = 32
HW = H * W
L = 16
KV = K // L
NPT = K * C
CHUNK = 128
NCH = NPT // CHUNK
REM = NPT - NCH * CHUNK


def _body(out_hbm, ind_hbm, m_hbm, t_hbm, res_hbm,
          ind_v, idx_v, pred_v, m_v, t_v, part_v, sem):
    b = lax.axis_index("s")
    row0 = b * K

    pltpu.sync_copy(ind_hbm.at[pl.ds(row0, K)], ind_v)
    pltpu.sync_copy(m_hbm.at[pl.ds(row0 * C, NPT)], m_v)
    pltpu.sync_copy(t_hbm.at[pl.ds(row0 * C, NPT)], t_v)

    base_b = b * (C * HW)
    inds = [ind_v[pl.ds(kv * L, L)] for kv in range(KV)]
    for c in range(C):
        off = base_b + c * HW
        for kv in range(KV):
            idx_v[pl.ds((kv * C + c) * L, L)] = inds[kv] + off

    copies = []
    for j in range(NCH):
        copies.append(pltpu.async_copy(
            out_hbm.at[idx_v.at[pl.ds(j * CHUNK, CHUNK)]],
            pred_v.at[pl.ds(j * CHUNK, CHUNK)], sem))
    copies.append(pltpu.async_copy(
        out_hbm.at[idx_v.at[pl.ds(NCH * CHUNK, REM)]],
        pred_v.at[pl.ds(NCH * CHUNK, REM)], sem))
    for cp in copies:
        cp.wait()

    lane_c = jnp.arange(L, dtype=jnp.int32) * C
    acc = jnp.zeros((L,), jnp.float32)
    mac = jnp.zeros((L,), jnp.float32)
    for kv in range(KV):
        kbase = kv * L * C
        for c in range(C):
            p = pred_v[pl.ds((kv * C + c) * L, L)]
            kidx = lane_c + (kbase + c)
            m = plsc.load_gather(m_v, [kidx])
            tg = plsc.load_gather(t_v, [kidx])
            acc = acc + jnp.abs(p * m - tg * m)
            mac = mac + m
    part_v[pl.ds(0, L)] = acc
    part_v[pl.ds(L, L)] = mac

    pltpu.sync_copy(part_v, res_hbm.at[b])


@jax.jit
def kernel(output, mask, ind, target):
    out_flat = output.reshape(-1)
    ind_flat = ind.astype(jnp.int32).reshape(-1)
    m_flat = mask.astype(jnp.float32).reshape(-1)
    t_flat = target.astype(jnp.float32).reshape(-1)

    mesh = plsc.VectorSubcoreMesh(
        core_axis_name="c", subcore_axis_name="s", num_cores=1)
    run = functools.partial(
        pl.kernel,
        out_type=jax.ShapeDtypeStruct((B, 2 * L), jnp.float32),
        mesh=mesh,
        scratch_types=[
            pltpu.VMEM((K,), jnp.int32),
            pltpu.VMEM((NPT,), jnp.int32),
            pltpu.VMEM((NPT,), jnp.float32),
            pltpu.VMEM((NPT,), jnp.float32),
            pltpu.VMEM((NPT,), jnp.float32),
            pltpu.VMEM((2 * L,), jnp.float32),
            pltpu.SemaphoreType.DMA,
        ],
        compiler_params=pltpu.CompilerParams(needs_layout_passes=False),
    )(_body)
    res = run(out_flat, ind_flat, m_flat, t_flat)
    return jnp.sum(res[:, :L]) / (jnp.sum(res[:, L:]) + 0.0001)


# speedup vs baseline: 3.3753x; 1.0907x over previous
import functools

import jax
import jax.numpy as jnp
from jax import lax
from jax.experimental import pallas as pl
from jax.experimental.pallas import tpu as pltpu
from jax.experimental.pallas import tpu_sc as plsc

B, C, H, W = 16, 34, 128, 128
K = 32
HW = H * W
L = 16
NC, NS = 2, 16
NPT = L * C
CHUNK = 128
NCH = NPT // CHUNK
REM = NPT - NCH * CHUNK


def _body(out_hbm, ind_hbm, m_hbm, t_hbm, res_hbm,
          ind_v, idx_v, pred_v, m_v, t_v, part_v, sem, sem2):
    cid = lax.axis_index("c")
    b = lax.axis_index("s")
    k0 = cid * L

    pltpu.sync_copy(ind_hbm.at[b, pl.ds(k0, L)], ind_v)
    mcp = pltpu.async_copy(m_hbm.at[b, pl.ds(k0, L)], m_v, sem2)
    tcp = pltpu.async_copy(t_hbm.at[b, pl.ds(k0, L)], t_v, sem2)

    base_b = b * (C * HW)
    ind_vec = ind_v[...]
    for c in range(C):
        idx_v[pl.ds(c * L, L)] = ind_vec + (base_b + c * HW)

    copies = []
    for j in range(NCH):
        copies.append(pltpu.async_copy(
            out_hbm.at[idx_v.at[pl.ds(j * CHUNK, CHUNK)]],
            pred_v.at[pl.ds(j * CHUNK, CHUNK)], sem))
    copies.append(pltpu.async_copy(
        out_hbm.at[idx_v.at[pl.ds(NCH * CHUNK, REM)]],
        pred_v.at[pl.ds(NCH * CHUNK, REM)], sem))
    for cp in copies:
        cp.wait()
    mcp.wait()
    tcp.wait()

    lane = jnp.arange(L, dtype=jnp.int32)
    cvec = jnp.zeros((L,), jnp.int32)
    acc = jnp.zeros((L,), jnp.float32)
    mac = jnp.zeros((L,), jnp.float32)
    for c in range(C):
        p = pred_v[pl.ds(c * L, L)]
        m = plsc.load_gather(m_v, [lane, cvec + c])
        tg = plsc.load_gather(t_v, [lane, cvec + c])
        acc = acc + jnp.abs(p * m - tg * m)
        mac = mac + m
    part_v[pl.ds(0, L)] = acc
    part_v[pl.ds(L, L)] = mac

    pltpu.sync_copy(part_v, res_hbm.at[cid * NS + b])


@jax.jit
def kernel(output, mask, ind, target):
    out_flat = output.reshape(-1)
    ind32 = ind.astype(jnp.int32)

    mesh = plsc.VectorSubcoreMesh(core_axis_name="c", subcore_axis_name="s")
    run = functools.partial(
        pl.kernel,
        out_type=jax.ShapeDtypeStruct((NC * NS, 2 * L), jnp.float32),
        mesh=mesh,
        scratch_types=[
            pltpu.VMEM((L,), jnp.int32),
            pltpu.VMEM((NPT,), jnp.int32),
            pltpu.VMEM((NPT,), jnp.float32),
            pltpu.VMEM((L, C), jnp.float32),
            pltpu.VMEM((L, C), jnp.float32),
            pltpu.VMEM((2 * L,), jnp.float32),
            pltpu.SemaphoreType.DMA,
            pltpu.SemaphoreType.DMA,
        ],
        compiler_params=pltpu.CompilerParams(needs_layout_passes=False),
    )(_body)
    res = run(out_flat, ind32, mask, target)
    return jnp.sum(res[:, :L]) / (jnp.sum(res[:, L:]) + 0.0001)


# speedup vs baseline: 3.4135x; 1.0113x over previous
import functools

import jax
import jax.numpy as jnp
from jax import lax
from jax.experimental import pallas as pl
from jax.experimental.pallas import tpu as pltpu
from jax.experimental.pallas import tpu_sc as plsc

B, C, H, W = 16, 34, 128, 128
K = 32
HW = H * W
L = 16
NC, NS = 2, 16
NPT = L * C
CHUNK = 128
NCH = NPT // CHUNK
REM = NPT - NCH * CHUNK


def _body(out_hbm, ind_hbm, m_hbm, t_hbm, res_hbm,
          ind_v, idx_v, pred_v, m_v, t_v, part_v, sem, sem2):
    cid = lax.axis_index("c")
    b = lax.axis_index("s")
    k0 = cid * L

    pltpu.sync_copy(ind_hbm.at[b, pl.ds(k0, L)], ind_v)
    mcp = pltpu.async_copy(m_hbm.at[b, pl.ds(k0, L)], m_v, sem2)
    tcp = pltpu.async_copy(t_hbm.at[b, pl.ds(k0, L)], t_v, sem2)

    base_b = b * (C * HW)
    ind_vec = ind_v[...]

    def build(c, _):
        idx_v[pl.ds(c * L, L)] = ind_vec + (base_b + c * HW)
        return 0

    lax.fori_loop(0, C, build, 0)

    copies = []
    for j in range(NCH):
        copies.append(pltpu.async_copy(
            out_hbm.at[idx_v.at[pl.ds(j * CHUNK, CHUNK)]],
            pred_v.at[pl.ds(j * CHUNK, CHUNK)], sem))
    copies.append(pltpu.async_copy(
        out_hbm.at[idx_v.at[pl.ds(NCH * CHUNK, REM)]],
        pred_v.at[pl.ds(NCH * CHUNK, REM)], sem))
    for cp in copies:
        cp.wait()
    mcp.wait()
    tcp.wait()

    lane = jnp.arange(L, dtype=jnp.int32)
    cvec = jnp.zeros((L,), jnp.int32)

    def step(c, carry):
        acc, mac = carry
        p = pred_v[pl.ds(c * L, L)]
        m = plsc.load_gather(m_v, [lane, cvec + c])
        tg = plsc.load_gather(t_v, [lane, cvec + c])
        return acc + jnp.abs(p * m - tg * m), mac + m

    acc, mac = lax.fori_loop(
        0, C, step,
        (jnp.zeros((L,), jnp.float32), jnp.zeros((L,), jnp.float32)))
    part_v[pl.ds(0, L)] = acc
    part_v[pl.ds(L, L)] = mac

    pltpu.sync_copy(part_v, res_hbm.at[cid * NS + b])


@jax.jit
def kernel(output, mask, ind, target):
    out_flat = output.reshape(-1)
    ind32 = ind.astype(jnp.int32)

    mesh = plsc.VectorSubcoreMesh(core_axis_name="c", subcore_axis_name="s")
    run = functools.partial(
        pl.kernel,
        out_type=jax.ShapeDtypeStruct((NC * NS, 2 * L), jnp.float32),
        mesh=mesh,
        scratch_types=[
            pltpu.VMEM((L,), jnp.int32),
            pltpu.VMEM((NPT,), jnp.int32),
            pltpu.VMEM((NPT,), jnp.float32),
            pltpu.VMEM((L, C), jnp.float32),
            pltpu.VMEM((L, C), jnp.float32),
            pltpu.VMEM((2 * L,), jnp.float32),
            pltpu.SemaphoreType.DMA,
            pltpu.SemaphoreType.DMA,
        ],
        compiler_params=pltpu.CompilerParams(needs_layout_passes=False),
    )(_body)
    res = run(out_flat, ind32, mask, target)
    return jnp.sum(res[:, :L]) / (jnp.sum(res[:, L:]) + 0.0001)


# speedup vs baseline: 3.4315x; 1.0053x over previous
import functools

import jax
import jax.numpy as jnp
from jax import lax
from jax.experimental import pallas as pl
from jax.experimental.pallas import tpu as pltpu
from jax.experimental.pallas import tpu_sc as plsc

B, C, H, W = 16, 34, 128, 128
K = 32
HW = H * W
L = 16
KV = K // L
NPT = K * C
CHUNK = 128
NCH = NPT // CHUNK
REM = NPT - NCH * CHUNK


def _body(out_hbm, ind_hbm, m_hbm, t_hbm, res_hbm,
          ind_v, idx_v, pred_v, m_v, t_v, part_v, sem, sem2):
    b = lax.axis_index("s")

    pltpu.sync_copy(ind_hbm.at[b], ind_v)
    mcp = pltpu.async_copy(m_hbm.at[b], m_v, sem2)
    tcp = pltpu.async_copy(t_hbm.at[b], t_v, sem2)

    base_b = b * (C * HW)
    ind0 = ind_v[pl.ds(0, L)]
    ind1 = ind_v[pl.ds(L, L)]

    def build(c, _):
        off = base_b + c * HW
        idx_v[pl.ds(c * L, L)] = ind0 + off
        idx_v[pl.ds((C + c) * L, L)] = ind1 + off
        return 0

    lax.fori_loop(0, C, build, 0)

    copies = []
    for j in range(NCH):
        copies.append(pltpu.async_copy(
            out_hbm.at[idx_v.at[pl.ds(j * CHUNK, CHUNK)]],
            pred_v.at[pl.ds(j * CHUNK, CHUNK)], sem))
    copies.append(pltpu.async_copy(
        out_hbm.at[idx_v.at[pl.ds(NCH * CHUNK, REM)]],
        pred_v.at[pl.ds(NCH * CHUNK, REM)], sem))
    for cp in copies:
        cp.wait()
    mcp.wait()
    tcp.wait()

    lane = jnp.arange(L, dtype=jnp.int32)
    cvec = jnp.zeros((L,), jnp.int32)

    def step2(c, carry):
        acc, mac = carry
        p0 = pred_v[pl.ds(c * L, L)]
        m0 = plsc.load_gather(m_v, [lane, cvec + c])
        t0 = plsc.load_gather(t_v, [lane, cvec + c])
        acc = acc + jnp.abs(p0 * m0 - t0 * m0)
        mac = mac + m0
        p1 = pred_v[pl.ds((C + c) * L, L)]
        m1 = plsc.load_gather(m_v, [lane + L, cvec + c])
        t1 = plsc.load_gather(t_v, [lane + L, cvec + c])
        acc = acc + jnp.abs(p1 * m1 - t1 * m1)
        mac = mac + m1
        return acc, mac

    acc, mac = lax.fori_loop(
        0, C, step2,
        (jnp.zeros((L,), jnp.float32), jnp.zeros((L,), jnp.float32)))
    part_v[pl.ds(0, L)] = acc
    part_v[pl.ds(L, L)] = mac

    pltpu.sync_copy(part_v, res_hbm.at[b])


@jax.jit
def kernel(output, mask, ind, target):
    out_flat = output.reshape(-1)
    ind32 = ind.astype(jnp.int32)

    mesh = plsc.VectorSubcoreMesh(
        core_axis_name="c", subcore_axis_name="s", num_cores=1)
    run = functools.partial(
        pl.kernel,
        out_type=jax.ShapeDtypeStruct((B, 2 * L), jnp.float32),
        mesh=mesh,
        scratch_types=[
            pltpu.VMEM((K,), jnp.int32),
            pltpu.VMEM((NPT,), jnp.int32),
            pltpu.VMEM((NPT,), jnp.float32),
            pltpu.VMEM((K, C), jnp.float32),
            pltpu.VMEM((K, C), jnp.float32),
            pltpu.VMEM((2 * L,), jnp.float32),
            pltpu.SemaphoreType.DMA,
            pltpu.SemaphoreType.DMA,
        ],
        compiler_params=pltpu.CompilerParams(needs_layout_passes=False),
    )(_body)
    res = run(out_flat, ind32, mask, target)
    return jnp.sum(res[:, :L]) / (jnp.sum(res[:, L:]) + 0.0001)


# speedup vs baseline: 3.7465x; 1.0918x over previous
import functools

import jax
import jax.numpy as jnp
from jax import lax
from jax.experimental import pallas as pl
from jax.experimental.pallas import tpu as pltpu
from jax.experimental.pallas import tpu_sc as plsc

B, C, H, W = 16, 34, 128, 128
K = 32
HW = H * W
L = 16
NPT = K * C
CHUNK = 128
NCH = NPT // CHUNK
REM = NPT - NCH * CHUNK


def _body(out_hbm, ind_hbm, m_hbm, t_hbm, part_hbm, res_hbm,
          ind_v, idx_v, pred_v, m_v, t_v, part_v, all_v, out_v, sem, sem2):
    b = lax.axis_index("s")

    pltpu.sync_copy(ind_hbm.at[b], ind_v)
    mcp = pltpu.async_copy(m_hbm.at[b], m_v, sem2)
    tcp = pltpu.async_copy(t_hbm.at[b], t_v, sem2)

    base_b = b * (C * HW)
    ind0 = ind_v[pl.ds(0, L)]
    ind1 = ind_v[pl.ds(L, L)]

    def build(c, _):
        off = base_b + c * HW
        idx_v[pl.ds(c * L, L)] = ind0 + off
        idx_v[pl.ds((C + c) * L, L)] = ind1 + off
        return 0

    lax.fori_loop(0, C, build, 0)

    copies = []
    for j in range(NCH):
        copies.append(pltpu.async_copy(
            out_hbm.at[idx_v.at[pl.ds(j * CHUNK, CHUNK)]],
            pred_v.at[pl.ds(j * CHUNK, CHUNK)], sem))
    copies.append(pltpu.async_copy(
        out_hbm.at[idx_v.at[pl.ds(NCH * CHUNK, REM)]],
        pred_v.at[pl.ds(NCH * CHUNK, REM)], sem))
    for cp in copies:
        cp.wait()
    mcp.wait()
    tcp.wait()

    lane = jnp.arange(L, dtype=jnp.int32)
    cvec = jnp.zeros((L,), jnp.int32)

    def step(c, carry):
        acc, mac = carry
        p0 = pred_v[pl.ds(c * L, L)]
        m0 = plsc.load_gather(m_v, [lane, cvec + c])
        t0 = plsc.load_gather(t_v, [lane, cvec + c])
        acc = acc + jnp.abs(p0 * m0 - t0 * m0)
        mac = mac + m0
        p1 = pred_v[pl.ds((C + c) * L, L)]
        m1 = plsc.load_gather(m_v, [lane + L, cvec + c])
        t1 = plsc.load_gather(t_v, [lane + L, cvec + c])
        acc = acc + jnp.abs(p1 * m1 - t1 * m1)
        mac = mac + m1
        return acc, mac

    acc, mac = lax.fori_loop(
        0, C, step,
        (jnp.zeros((L,), jnp.float32), jnp.zeros((L,), jnp.float32)))
    part_v[pl.ds(0, L)] = acc
    part_v[pl.ds(L, L)] = mac

    pltpu.sync_copy(part_v, part_hbm.at[b])
    plsc.subcore_barrier()

    @pl.when(b == 0)
    def _():
        pltpu.sync_copy(part_hbm, all_v)

        def red(i, carry):
            a, m = carry
            return a + all_v[i, pl.ds(0, L)], m + all_v[i, pl.ds(L, L)]

        a, m = lax.fori_loop(
            0, B, red,
            (jnp.zeros((L,), jnp.float32), jnp.zeros((L,), jnp.float32)))
        num = jnp.full((L,), jnp.sum(a), jnp.float32)
        den = jnp.full((L,), jnp.sum(m), jnp.float32) + 0.0001
        out_v[...] = num / den
        pltpu.sync_copy(out_v.at[pl.ds(0, 1)], res_hbm)


@jax.jit
def kernel(output, mask, ind, target):
    out_flat = output.reshape(-1)
    ind32 = ind.astype(jnp.int32)

    mesh = plsc.VectorSubcoreMesh(
        core_axis_name="c", subcore_axis_name="s", num_cores=1)
    run = functools.partial(
        pl.kernel,
        out_type=(jax.ShapeDtypeStruct((B, 2 * L), jnp.float32),
                  jax.ShapeDtypeStruct((1,), jnp.float32)),
        mesh=mesh,
        scratch_types=[
            pltpu.VMEM((K,), jnp.int32),
            pltpu.VMEM((NPT,), jnp.int32),
            pltpu.VMEM((NPT,), jnp.float32),
            pltpu.VMEM((K, C), jnp.float32),
            pltpu.VMEM((K, C), jnp.float32),
            pltpu.VMEM((2 * L,), jnp.float32),
            pltpu.VMEM((B, 2 * L), jnp.float32),
            pltpu.VMEM((L,), jnp.float32),
            pltpu.SemaphoreType.DMA,
            pltpu.SemaphoreType.DMA,
        ],
        compiler_params=pltpu.CompilerParams(needs_layout_passes=False),
    )(_body)
    _, res = run(out_flat, ind32, mask, target)
    return res.reshape(())
